# baseline (speedup 1.0000x reference)
import functools

import jax
import jax.numpy as jnp
from jax.experimental import pallas as pl
from jax.experimental.pallas import tpu as pltpu


def _gcn_kernel(x_ref, w1_ref, w2_ref, a_ref, o_ref, z_ref, h_ref, r_ref,
                *, bm, nblk, nret):
    p = pl.program_id(0)
    i = pl.program_id(1)

    @pl.when((p == 0) & (i == 0))
    def _():
        z_ref[...] = jnp.dot(
            x_ref[...].astype(jnp.bfloat16),
            w1_ref[...].astype(jnp.bfloat16),
            preferred_element_type=jnp.float32,
        ).astype(jnp.bfloat16)

    @pl.when((p == 1) & (i == 0))
    def _():
        z_ref[...] = jnp.dot(
            h_ref[...],
            w2_ref[...].astype(jnp.bfloat16),
            preferred_element_type=jnp.float32,
        ).astype(jnp.bfloat16)

    @pl.when(p == 0)
    def _():
        a16 = a_ref[...].astype(jnp.bfloat16)
        acc = jnp.dot(a16, z_ref[...], preferred_element_type=jnp.float32)
        j = nblk - 1 - i
        h_ref[pl.ds(j * bm, bm), :] = acc.astype(jnp.bfloat16)

        @pl.when((j >= 1) & (j <= nret))
        def _():
            r_ref[pl.ds((j - 1) * bm, bm), :] = a16

    use_ret = (p == 1) & (i >= 1) & (i <= nret)

    @pl.when((p == 1) & ~use_ret)
    def _():
        o_ref[...] = jnp.dot(a_ref[...].astype(jnp.bfloat16), z_ref[...],
                             preferred_element_type=jnp.float32)

    @pl.when(use_ret)
    def _():
        o_ref[...] = jnp.dot(r_ref[pl.ds((i - 1) * bm, bm), :], z_ref[...],
                             preferred_element_type=jnp.float32)


def kernel(X, A_hat, W1, W2):
    n = A_hat.shape[0]
    d = W1.shape[1]
    bm = 200 if n % 200 == 0 else n
    nblk = n // bm
    nret = 6 if nblk > 8 else 0

    def a_map(p, i):
        fwd = (i > nret).astype(jnp.int32) * i
        return ((1 - p) * (nblk - 1 - i) + p * fwd, 0)

    return pl.pallas_call(
        functools.partial(_gcn_kernel, bm=bm, nblk=nblk, nret=nret),
        grid=(2, nblk),
        in_specs=[
            pl.BlockSpec((n, d), lambda p, i: (0, 0)),
            pl.BlockSpec((d, d), lambda p, i: (0, 0)),
            pl.BlockSpec((d, d), lambda p, i: (0, 0)),
            pl.BlockSpec((bm, n), a_map),
        ],
        out_specs=pl.BlockSpec((bm, d), lambda p, i: (p * i, 0)),
        out_shape=jax.ShapeDtypeStruct((n, d), jnp.float32),
        scratch_shapes=[
            pltpu.VMEM((n, d), jnp.bfloat16),
            pltpu.VMEM((n, d), jnp.bfloat16),
            pltpu.VMEM((max(nret, 1) * bm, n), jnp.bfloat16),
        ],
    )(X, W1, W2, A_hat)


# speedup vs baseline: 1.0126x; 1.0126x over previous
import functools

import jax
import jax.numpy as jnp
from jax.experimental import pallas as pl
from jax.experimental.pallas import tpu as pltpu


def _gcn_kernel(x_ref, w1_ref, w2_ref, a_ref, o_ref, z_ref, h_ref, *, bm):
    p = pl.program_id(0)
    i = pl.program_id(1)

    @pl.when((p == 0) & (i == 0))
    def _():
        z_ref[...] = jnp.dot(x_ref[...], w1_ref[...],
                             preferred_element_type=jnp.float32)

    @pl.when((p == 1) & (i == 0))
    def _():
        z_ref[...] = jnp.dot(h_ref[...], w2_ref[...],
                             preferred_element_type=jnp.float32)

    acc = jnp.dot(a_ref[...], z_ref[...], preferred_element_type=jnp.float32)

    @pl.when(p == 0)
    def _():
        nblk = pl.num_programs(1)
        j = nblk - 1 - i
        h_ref[pl.ds(j * bm, bm), :] = acc

    @pl.when(p == 1)
    def _():
        o_ref[...] = acc


def kernel(X, A_hat, W1, W2):
    n = A_hat.shape[0]
    d = W1.shape[1]
    bm = 400 if n % 400 == 0 else n
    nblk = n // bm

    def a_map(p, i):
        return ((1 - p) * (nblk - 1 - i) + p * i, 0)

    return pl.pallas_call(
        functools.partial(_gcn_kernel, bm=bm),
        grid=(2, nblk),
        in_specs=[
            pl.BlockSpec((n, d), lambda p, i: (0, 0)),
            pl.BlockSpec((d, d), lambda p, i: (0, 0)),
            pl.BlockSpec((d, d), lambda p, i: (0, 0)),
            pl.BlockSpec((bm, n), a_map),
        ],
        out_specs=pl.BlockSpec((bm, d), lambda p, i: (p * i, 0)),
        out_shape=jax.ShapeDtypeStruct((n, d), jnp.float32),
        scratch_shapes=[
            pltpu.VMEM((n, d), jnp.float32),
            pltpu.VMEM((n, d), jnp.float32),
        ],
    )(X, W1, W2, A_hat)


# speedup vs baseline: 1.0210x; 1.0084x over previous
import functools

import jax
import jax.numpy as jnp
from jax.experimental import pallas as pl
from jax.experimental.pallas import tpu as pltpu


def _gcn_kernel(x_ref, w1_ref, w2_ref, a_ref, o_ref, z_ref, h_ref, r_ref,
                zb_ref, *, bm):
    p = pl.program_id(0)
    i = pl.program_id(1)

    @pl.when((p == 0) & (i == 0))
    def _():
        z_ref[...] = jnp.dot(x_ref[...], w1_ref[...],
                             preferred_element_type=jnp.float32)

    @pl.when((p == 1) & (i == 0))
    def _():
        z2 = jnp.dot(h_ref[...], w2_ref[...],
                     preferred_element_type=jnp.float32)
        z_ref[...] = z2
        zb_ref[...] = z2.astype(jnp.bfloat16)

    @pl.when(p == 0)
    def _():
        acc = jnp.dot(a_ref[...], z_ref[...],
                      preferred_element_type=jnp.float32)
        nblk = pl.num_programs(1)
        j = nblk - 1 - i
        h_ref[pl.ds(j * bm, bm), :] = acc

        @pl.when(j == 1)
        def _():
            r_ref[...] = a_ref[...].astype(jnp.bfloat16)

    @pl.when((p == 1) & (i != 1))
    def _():
        o_ref[...] = jnp.dot(a_ref[...], z_ref[...],
                             preferred_element_type=jnp.float32)

    @pl.when((p == 1) & (i == 1))
    def _():
        o_ref[...] = jnp.dot(r_ref[...], zb_ref[...],
                             preferred_element_type=jnp.float32)


def kernel(X, A_hat, W1, W2):
    n = A_hat.shape[0]
    d = W1.shape[1]
    bm = 400 if n % 400 == 0 else n
    nblk = n // bm

    def a_map(p, i):
        fwd = (i > 1).astype(jnp.int32) * i
        return ((1 - p) * (nblk - 1 - i) + p * fwd, 0)

    return pl.pallas_call(
        functools.partial(_gcn_kernel, bm=bm),
        grid=(2, nblk),
        in_specs=[
            pl.BlockSpec((n, d), lambda p, i: (0, 0)),
            pl.BlockSpec((d, d), lambda p, i: (0, 0)),
            pl.BlockSpec((d, d), lambda p, i: (0, 0)),
            pl.BlockSpec((bm, n), a_map),
        ],
        out_specs=pl.BlockSpec((bm, d), lambda p, i: (p * i, 0)),
        out_shape=jax.ShapeDtypeStruct((n, d), jnp.float32),
        scratch_shapes=[
            pltpu.VMEM((n, d), jnp.float32),
            pltpu.VMEM((n, d), jnp.float32),
            pltpu.VMEM((bm, n), jnp.bfloat16),
            pltpu.VMEM((n, d), jnp.bfloat16),
        ],
    )(X, W1, W2, A_hat)
